# Initial kernel scaffold; baseline (speedup 1.0000x reference)
import jax
import jax.numpy as jnp
from jax.experimental import pallas as pl


def kernel(x, keys, values, W_dec, b_dec):
    raise NotImplementedError("write your pallas kernel here")


# speedup vs baseline: 1.3072x; 1.3072x over previous
import functools

import jax
import jax.numpy as jnp
from jax import lax
from jax.experimental import pallas as pl
from jax.experimental.pallas import tpu as pltpu
from jax.experimental.pallas import tpu_sc as plsc

_NUM_KEYS = 8192
_KEY_DIM = 32
_BATCH = 16384
_PAD_OUT = 16

_BB = 512
_KC = 2048
_NKC = _NUM_KEYS // _KC


def _argmax_decode_kernel(x_ref, keys_ref, values_ref, w_ref, b_ref,
                          idx_ref, dec_ref):
  x = x_ref[...]
  run_m = jnp.full((_BB, 1), -jnp.inf, dtype=jnp.float32)
  run_i = jnp.zeros((_BB, 1), dtype=jnp.int32)
  for c in range(_NKC):
    kc = keys_ref[c * _KC:(c + 1) * _KC, :]
    sim = lax.dot_general(x, kc, (((1,), (1,)), ((), ())),
                          preferred_element_type=jnp.float32)
    m = jnp.max(sim, axis=1, keepdims=True)
    iota = lax.broadcasted_iota(jnp.int32, (_BB, _KC), 1)
    li = jnp.min(jnp.where(sim == m, iota, jnp.int32(2**30)),
                 axis=1, keepdims=True) + c * _KC
    upd = m > run_m
    run_i = jnp.where(upd, li, run_i)
    run_m = jnp.where(upd, m, run_m)
  idx_ref[...] = run_i

  @pl.when(pl.program_id(0) == 0)
  def _():
    dec = lax.dot_general(values_ref[...], w_ref[...],
                          (((1,), (1,)), ((), ())),
                          preferred_element_type=jnp.float32)
    dec_ref[...] = dec + b_ref[...]


def _argmax_and_decode(x, keys, values, w_pad, b_pad):
  grid = _BATCH // _BB
  return pl.pallas_call(
      _argmax_decode_kernel,
      grid=(grid,),
      in_specs=[
          pl.BlockSpec((_BB, _KEY_DIM), lambda i: (i, 0)),
          pl.BlockSpec((_NUM_KEYS, _KEY_DIM), lambda i: (0, 0)),
          pl.BlockSpec((_NUM_KEYS, _KEY_DIM), lambda i: (0, 0)),
          pl.BlockSpec((_PAD_OUT, _KEY_DIM), lambda i: (0, 0)),
          pl.BlockSpec((1, _PAD_OUT), lambda i: (0, 0)),
      ],
      out_specs=[
          pl.BlockSpec((_BB, 1), lambda i: (i, 0)),
          pl.BlockSpec((_NUM_KEYS, _PAD_OUT), lambda i: (0, 0)),
      ],
      out_shape=[
          jax.ShapeDtypeStruct((_BATCH, 1), jnp.int32),
          jax.ShapeDtypeStruct((_NUM_KEYS, _PAD_OUT), jnp.float32),
      ],
  )(x, keys, values, w_pad, b_pad)


def _make_sc_gather():
  info = plsc.get_sparse_core_info()
  nc, ns = info.num_cores, info.num_subcores
  nw = nc * ns
  b_per_w = _BATCH // nw
  mesh = plsc.VectorSubcoreMesh(core_axis_name="c", subcore_axis_name="s")

  @functools.partial(
      pl.kernel,
      out_type=jax.ShapeDtypeStruct((_BATCH, _PAD_OUT), jnp.float32),
      mesh=mesh,
      scratch_types=[
          pltpu.VMEM((b_per_w,), jnp.int32),
          pltpu.VMEM((b_per_w, _PAD_OUT), jnp.float32),
          pltpu.SemaphoreType.DMA,
      ],
      compiler_params=pltpu.CompilerParams(use_tc_tiling_on_sc=False),
  )
  def gather(table_hbm, idx_hbm, out_hbm, idx_v, rows_v, sem):
    wid = lax.axis_index("s") * nc + lax.axis_index("c")
    base = wid * b_per_w
    pltpu.sync_copy(idx_hbm.at[pl.ds(base, b_per_w)], idx_v)
    pltpu.async_copy(table_hbm.at[idx_v], rows_v, sem).wait()
    pltpu.sync_copy(rows_v, out_hbm.at[pl.ds(base, b_per_w)])

  return gather


_sc_gather = None


def kernel(x, keys, values, W_dec, b_dec):
  global _sc_gather
  if _sc_gather is None:
    _sc_gather = _make_sc_gather()
  w_pad = jnp.zeros((_PAD_OUT, _KEY_DIM), jnp.float32).at[:W_dec.shape[0], :].set(W_dec)
  b_pad = jnp.zeros((1, _PAD_OUT), jnp.float32).at[0, :b_dec.shape[0]].set(b_dec)
  idx, decoded = _argmax_and_decode(x, keys, values, w_pad, b_pad)
  gathered = _sc_gather(decoded, idx.reshape(-1))
  return gathered[:, :W_dec.shape[0]]


# speedup vs baseline: 1.8594x; 1.4224x over previous
import functools

import jax
import jax.numpy as jnp
from jax import lax
from jax.experimental import pallas as pl
from jax.experimental.pallas import tpu as pltpu
from jax.experimental.pallas import tpu_sc as plsc

_NUM_KEYS = 8192
_KEY_DIM = 32
_BATCH = 16384
_PAD_OUT = 16

_BB = 512
_KC = 2048
_NKC = _NUM_KEYS // _KC


def _argmax_decode_kernel(x_ref, keys_ref, values_ref, w_ref, b_ref,
                          idx_ref, dec_ref):
  x = x_ref[...]
  sim = lax.dot_general(x, keys_ref[...], (((1,), (1,)), ((), ())),
                        preferred_element_type=jnp.float32)
  idx_ref[...] = jnp.argmax(sim, axis=1).astype(jnp.int32).reshape(_BB, 1)

  @pl.when(pl.program_id(0) == 0)
  def _():
    dec = lax.dot_general(values_ref[...], w_ref[...],
                          (((1,), (1,)), ((), ())),
                          preferred_element_type=jnp.float32)
    dec_ref[...] = dec + b_ref[...]


def _argmax_and_decode(x, keys, values, w_pad, b_pad):
  grid = _BATCH // _BB
  return pl.pallas_call(
      _argmax_decode_kernel,
      grid=(grid,),
      in_specs=[
          pl.BlockSpec((_BB, _KEY_DIM), lambda i: (i, 0)),
          pl.BlockSpec((_NUM_KEYS, _KEY_DIM), lambda i: (0, 0)),
          pl.BlockSpec((_NUM_KEYS, _KEY_DIM), lambda i: (0, 0)),
          pl.BlockSpec((_PAD_OUT, _KEY_DIM), lambda i: (0, 0)),
          pl.BlockSpec((1, _PAD_OUT), lambda i: (0, 0)),
      ],
      out_specs=[
          pl.BlockSpec((_BB, 1), lambda i: (i, 0)),
          pl.BlockSpec((_NUM_KEYS, _PAD_OUT), lambda i: (0, 0)),
      ],
      out_shape=[
          jax.ShapeDtypeStruct((_BATCH, 1), jnp.int32),
          jax.ShapeDtypeStruct((_NUM_KEYS, _PAD_OUT), jnp.float32),
      ],
  )(x, keys, values, w_pad, b_pad)


def _make_sc_gather():
  info = plsc.get_sparse_core_info()
  nc, ns = info.num_cores, info.num_subcores
  nw = nc * ns
  b_per_w = _BATCH // nw
  mesh = plsc.VectorSubcoreMesh(core_axis_name="c", subcore_axis_name="s")

  @functools.partial(
      pl.kernel,
      out_type=jax.ShapeDtypeStruct((_BATCH, _PAD_OUT), jnp.float32),
      mesh=mesh,
      scratch_types=[
          pltpu.VMEM((b_per_w,), jnp.int32),
          pltpu.VMEM((b_per_w, _PAD_OUT), jnp.float32),
          pltpu.SemaphoreType.DMA,
      ],
      compiler_params=pltpu.CompilerParams(use_tc_tiling_on_sc=False),
  )
  def gather(table_hbm, idx_hbm, out_hbm, idx_v, rows_v, sem):
    wid = lax.axis_index("s") * nc + lax.axis_index("c")
    base = wid * b_per_w
    pltpu.sync_copy(idx_hbm.at[pl.ds(base, b_per_w)], idx_v)
    pltpu.async_copy(table_hbm.at[idx_v], rows_v, sem).wait()
    pltpu.sync_copy(rows_v, out_hbm.at[pl.ds(base, b_per_w)])

  return gather


_sc_gather = None


def kernel(x, keys, values, W_dec, b_dec):
  global _sc_gather
  if _sc_gather is None:
    _sc_gather = _make_sc_gather()
  w_pad = jnp.zeros((_PAD_OUT, _KEY_DIM), jnp.float32).at[:W_dec.shape[0], :].set(W_dec)
  b_pad = jnp.zeros((1, _PAD_OUT), jnp.float32).at[0, :b_dec.shape[0]].set(b_dec)
  idx, decoded = _argmax_and_decode(x, keys, values, w_pad, b_pad)
  gathered = _sc_gather(decoded, idx.reshape(-1))
  return gathered[:, :W_dec.shape[0]]


# speedup vs baseline: 1.9283x; 1.0371x over previous
import functools

import jax
import jax.numpy as jnp
from jax import lax
from jax.experimental import pallas as pl
from jax.experimental.pallas import tpu as pltpu
from jax.experimental.pallas import tpu_sc as plsc

_NUM_KEYS = 8192
_KEY_DIM = 32
_BATCH = 16384
_OUT_DIM = 10
_PAD_OUT = 16

_BB = 512


def _argmax_decode_kernel(x_ref, keys_ref, values_ref, w_ref, b_ref,
                          idx_ref, dec_ref):
  sim = lax.dot_general(x_ref[...], keys_ref[...], (((1,), (1,)), ((), ())),
                        preferred_element_type=jnp.float32)
  idx_ref[...] = jnp.argmax(sim, axis=1).astype(jnp.int32)

  @pl.when(pl.program_id(0) == 0)
  def _():
    dec = lax.dot_general(values_ref[...], w_ref[...],
                          (((1,), (1,)), ((), ())),
                          preferred_element_type=jnp.float32)
    dec_ref[...] = dec + b_ref[...]


def _argmax_and_decode(x, keys, values, w, b):
  grid = _BATCH // _BB
  return pl.pallas_call(
      _argmax_decode_kernel,
      grid=(grid,),
      in_specs=[
          pl.BlockSpec((_BB, _KEY_DIM), lambda i: (i, 0)),
          pl.BlockSpec((_NUM_KEYS, _KEY_DIM), lambda i: (0, 0)),
          pl.BlockSpec((_NUM_KEYS, _KEY_DIM), lambda i: (0, 0)),
          pl.BlockSpec((_OUT_DIM, _KEY_DIM), lambda i: (0, 0)),
          pl.BlockSpec((1, _OUT_DIM), lambda i: (0, 0)),
      ],
      out_specs=[
          pl.BlockSpec((_BB,), lambda i: (i,)),
          pl.BlockSpec((_NUM_KEYS, _OUT_DIM), lambda i: (0, 0)),
      ],
      out_shape=[
          jax.ShapeDtypeStruct((_BATCH,), jnp.int32),
          jax.ShapeDtypeStruct((_NUM_KEYS, _OUT_DIM), jnp.float32),
      ],
  )(x, keys, values, w, b)


def _make_sc_gather():
  info = plsc.get_sparse_core_info()
  nc, ns = info.num_cores, info.num_subcores
  nw = nc * ns
  b_per_w = _BATCH // nw
  mesh = plsc.VectorSubcoreMesh(core_axis_name="c", subcore_axis_name="s")

  @functools.partial(
      pl.kernel,
      out_type=jax.ShapeDtypeStruct((_BATCH, _OUT_DIM), jnp.float32),
      mesh=mesh,
      scratch_types=[
          pltpu.VMEM((b_per_w,), jnp.int32),
          pltpu.VMEM((b_per_w, _OUT_DIM), jnp.float32),
          pltpu.SemaphoreType.DMA,
      ],
      compiler_params=pltpu.CompilerParams(use_tc_tiling_on_sc=False),
  )
  def gather(table_hbm, idx_hbm, out_hbm, idx_v, rows_v, sem):
    wid = lax.axis_index("s") * nc + lax.axis_index("c")
    base = wid * b_per_w
    pltpu.sync_copy(idx_hbm.at[pl.ds(base, b_per_w)], idx_v)
    pltpu.async_copy(table_hbm.at[idx_v], rows_v, sem).wait()
    pltpu.sync_copy(rows_v, out_hbm.at[pl.ds(base, b_per_w)])

  return gather


_sc_gather = None


def kernel(x, keys, values, W_dec, b_dec):
  global _sc_gather
  if _sc_gather is None:
    _sc_gather = _make_sc_gather()
  idx, decoded = _argmax_and_decode(x, keys, values, W_dec,
                                    b_dec.reshape(1, _OUT_DIM))
  return _sc_gather(decoded, idx)


# speedup vs baseline: 1.9308x; 1.0013x over previous
import functools

import jax
import jax.numpy as jnp
from jax import lax
from jax.experimental import pallas as pl
from jax.experimental.pallas import tpu as pltpu
from jax.experimental.pallas import tpu_sc as plsc

_NUM_KEYS = 8192
_KEY_DIM = 32
_BATCH = 16384
_OUT_DIM = 10
_PAD_OUT = 16

_BB = 512


def _argmax_decode_kernel(x_ref, keys_ref, values_ref, w_ref, b_ref,
                          idx_ref, dec_ref):
  sim = lax.dot_general(x_ref[...], keys_ref[...], (((1,), (1,)), ((), ())),
                        preferred_element_type=jnp.float32)
  idx_ref[...] = jnp.argmax(sim, axis=1).astype(jnp.int32)

  @pl.when(pl.program_id(0) == 0)
  def _():
    dec = lax.dot_general(values_ref[...], w_ref[...],
                          (((1,), (1,)), ((), ())),
                          preferred_element_type=jnp.float32)
    dec = dec + b_ref[...]
    pad = jnp.zeros((_NUM_KEYS, _PAD_OUT - _OUT_DIM), jnp.float32)
    dec_ref[...] = jnp.concatenate([dec, pad], axis=1)


def _argmax_and_decode(x, keys, values, w, b):
  grid = _BATCH // _BB
  return pl.pallas_call(
      _argmax_decode_kernel,
      grid=(grid,),
      in_specs=[
          pl.BlockSpec((_BB, _KEY_DIM), lambda i: (i, 0)),
          pl.BlockSpec((_NUM_KEYS, _KEY_DIM), lambda i: (0, 0)),
          pl.BlockSpec((_NUM_KEYS, _KEY_DIM), lambda i: (0, 0)),
          pl.BlockSpec((_OUT_DIM, _KEY_DIM), lambda i: (0, 0)),
          pl.BlockSpec((1, _OUT_DIM), lambda i: (0, 0)),
      ],
      out_specs=[
          pl.BlockSpec((_BB,), lambda i: (i,)),
          pl.BlockSpec((_NUM_KEYS, _PAD_OUT), lambda i: (0, 0)),
      ],
      out_shape=[
          jax.ShapeDtypeStruct((_BATCH,), jnp.int32),
          jax.ShapeDtypeStruct((_NUM_KEYS, _PAD_OUT), jnp.float32),
      ],
  )(x, keys, values, w, b)


def _make_sc_gather():
  info = plsc.get_sparse_core_info()
  nc, ns = info.num_cores, info.num_subcores
  nw = nc * ns
  b_per_w = _BATCH // nw
  mesh = plsc.VectorSubcoreMesh(core_axis_name="c", subcore_axis_name="s")

  @functools.partial(
      pl.kernel,
      out_type=jax.ShapeDtypeStruct((_BATCH, _PAD_OUT), jnp.float32),
      mesh=mesh,
      scratch_types=[
          pltpu.VMEM((b_per_w,), jnp.int32),
          pltpu.VMEM((b_per_w, _PAD_OUT), jnp.float32),
          pltpu.SemaphoreType.DMA,
      ],
      compiler_params=pltpu.CompilerParams(use_tc_tiling_on_sc=False),
  )
  def gather(table_hbm, idx_hbm, out_hbm, idx_v, rows_v, sem):
    wid = lax.axis_index("s") * nc + lax.axis_index("c")
    base = wid * b_per_w
    pltpu.sync_copy(idx_hbm.at[pl.ds(base, b_per_w)], idx_v)
    pltpu.async_copy(table_hbm.at[idx_v], rows_v, sem).wait()
    pltpu.sync_copy(rows_v, out_hbm.at[pl.ds(base, b_per_w)])

  return gather


_sc_gather = None


def kernel(x, keys, values, W_dec, b_dec):
  global _sc_gather
  if _sc_gather is None:
    _sc_gather = _make_sc_gather()
  idx, decoded = _argmax_and_decode(x, keys, values, W_dec,
                                    b_dec.reshape(1, _OUT_DIM))
  return _sc_gather(decoded, idx)[:, :_OUT_DIM]
